# Initial kernel scaffold; baseline (speedup 1.0000x reference)
import jax
import jax.numpy as jnp
from jax.experimental import pallas as pl


def kernel(x, conv_w0, conv_b0, bn_g0, bn_b0, conv_w1, conv_b1, bn_g1, bn_b1, conv_w2, conv_b2, bn_g2, bn_b2, conv_w3, conv_b3, bn_g3, bn_b3, conv_w4, conv_b4, bn_g4, bn_b4, fuse_w, fuse_b, w_gate, training):
    raise NotImplementedError("write your pallas kernel here")


# speedup vs baseline: 1.9057x; 1.9057x over previous
import numpy as np
import jax
import jax.numpy as jnp
from jax import lax
from jax.experimental import pallas as pl

_D = 256
_T = 16
_NF = 8
_NS = 6
_IMG = 32
_NIMG = 128
_M = 16
_INV_BN = float(1.0 / np.sqrt(1.0 + 1e-05))


def _conv_stack_kernel(x_ref, w0, w1, w2, w3, w4, bb, fw, fb, out_ref):
    ws = (w0, w1, w2, w3, w4)
    h = x_ref[...]
    s = _IMG
    for l in range(5):
        rows = _M * (s // 2) * (s // 2)
        hp = h.reshape(_M, s, s // 2, 2 * _D)
        hp = hp.reshape(_M, s // 2, 2, s // 2, 2 * _D)
        p0 = hp[:, :, 0].reshape(rows, 2 * _D)
        p1 = hp[:, :, 1].reshape(rows, 2 * _D)
        patches = jnp.concatenate([p0, p1], axis=1)
        z = lax.dot_general(patches, ws[l][...], (((1,), (0,)), ((), ())),
                            preferred_element_type=jnp.float32)
        z = jnp.maximum(z + bb[l], 0.0)
        s //= 2
        h = z.reshape(_M, s, s, _D)
    h = h.reshape(_M, _D)
    fused = lax.dot_general(h, fw[...], (((1,), (1,)), ((), ())),
                            preferred_element_type=jnp.float32)
    out_ref[...] = fused + fb[...]


def _gate_kernel(fused_ref, cos_ref, sin_ref, wg_ref, gates_ref, load_ref):
    rows = []
    for b in range(8):
        fb = fused_ref[b]
        re = lax.dot_general(cos_ref[...], fb, (((0,), (0,)), ((), ())),
                             preferred_element_type=jnp.float32)
        im = lax.dot_general(sin_ref[...], fb, (((0,), (0,)), ((), ())),
                             preferred_element_type=jnp.float32)
        amp = jnp.sqrt(re * re + im * im)
        ampm = jnp.mean(amp, axis=1, keepdims=True)
        rows.append(jnp.sum(wg_ref[...] * ampm, axis=0, keepdims=True))
    logits = jnp.concatenate(rows, axis=0)

    idx = lax.broadcasted_iota(jnp.int32, (8, _NS), 1)
    m1 = jnp.max(logits, axis=1, keepdims=True)
    i1 = jnp.min(jnp.where(logits == m1, idx, _NS), axis=1, keepdims=True)
    mask1 = idx == i1
    l2 = jnp.where(mask1, -jnp.inf, logits)
    m2 = jnp.max(l2, axis=1, keepdims=True)
    i2 = jnp.min(jnp.where(l2 == m2, idx, _NS), axis=1, keepdims=True)
    mask2 = idx == i2
    e2 = jnp.exp(m2 - m1)
    g1 = 1.0 / (1.0 + e2)
    g2 = e2 / (1.0 + e2)
    gates = jnp.where(mask1, g1, 0.0) + jnp.where(mask2, g2, 0.0)
    gates_ref[...] = gates
    load_ref[...] = jnp.sum((gates > 0.0).astype(jnp.int32), axis=0,
                            keepdims=True)


def _dft_mats():
    t = np.arange(_T, dtype=np.float64)[:, None]
    f = np.arange(1, _NF + 1, dtype=np.float64)[None, :]
    ang = 2.0 * np.pi * t * f / _T
    scale = 1.0 / np.sqrt(_T)
    c = (np.cos(ang) * scale).astype(np.float32)
    s = (-np.sin(ang) * scale).astype(np.float32)
    return jnp.asarray(c), jnp.asarray(s)


def kernel(x, conv_w0, conv_b0, bn_g0, bn_b0, conv_w1, conv_b1, bn_g1, bn_b1,
           conv_w2, conv_b2, bn_g2, bn_b2, conv_w3, conv_b3, bn_g3, bn_b3,
           conv_w4, conv_b4, bn_g4, bn_b4, fuse_w, fuse_b, w_gate, training):
    del training
    xr = x.reshape(_NIMG, _IMG, _IMG, _D)
    conv_ws = [conv_w0, conv_w1, conv_w2, conv_w3, conv_w4]
    conv_bs = [conv_b0, conv_b1, conv_b2, conv_b3, conv_b4]
    bn_gs = [bn_g0, bn_g1, bn_g2, bn_g3, bn_g4]
    bn_bs = [bn_b0, bn_b1, bn_b2, bn_b3, bn_b4]
    wls = [w.transpose(2, 3, 1, 0).reshape(4 * _D, _D) * (g * _INV_BN)[None, :]
           for w, g in zip(conv_ws, bn_gs)]
    bbs = jnp.stack([b * g * _INV_BN + be
                     for b, g, be in zip(conv_bs, bn_gs, bn_bs)]).reshape(5, 1, _D)
    cmat, smat = _dft_mats()

    full = lambda shp: pl.BlockSpec(shp, lambda i: tuple(0 for _ in shp))
    fused = pl.pallas_call(
        _conv_stack_kernel,
        grid=(_NIMG // _M,),
        in_specs=[
            pl.BlockSpec((_M, _IMG, _IMG, _D), lambda i: (i, 0, 0, 0)),
            full((4 * _D, _D)), full((4 * _D, _D)), full((4 * _D, _D)),
            full((4 * _D, _D)), full((4 * _D, _D)),
            full((5, 1, _D)),
            full((_D, _D)), full((1, _D)),
        ],
        out_specs=pl.BlockSpec((_M, _D), lambda i: (i, 0)),
        out_shape=jax.ShapeDtypeStruct((_NIMG, _D), jnp.float32),
    )(xr, wls[0], wls[1], wls[2], wls[3], wls[4], bbs,
      fuse_w, fuse_b.reshape(1, _D))

    gates, load = pl.pallas_call(
        _gate_kernel,
        in_specs=[
            pl.BlockSpec((8, _T, _D), lambda: (0, 0, 0)),
            pl.BlockSpec((_T, _NF), lambda: (0, 0)),
            pl.BlockSpec((_T, _NF), lambda: (0, 0)),
            pl.BlockSpec((_NF, _NS), lambda: (0, 0)),
        ],
        out_specs=[
            pl.BlockSpec((8, _NS), lambda: (0, 0)),
            pl.BlockSpec((1, _NS), lambda: (0, 0)),
        ],
        out_shape=[
            jax.ShapeDtypeStruct((8, _NS), jnp.float32),
            jax.ShapeDtypeStruct((1, _NS), jnp.int32),
        ],
    )(fused.reshape(8, _T, _D), cmat, smat, w_gate)
    return gates, load.reshape(_NS)
